# baseline (speedup 1.0000x reference)
import functools

import jax
import jax.numpy as jnp
from jax import lax
from jax.experimental import pallas as pl
from jax.experimental.pallas import tpu as pltpu
from jax.experimental.pallas import tpu_sc as plsc

EMBEDDING_DIM = 128
CHUNK = 64
WBUF = 5


def _make_gather(num_rows: int):
    info = plsc.get_sparse_core_info()
    nc, ns = info.num_cores, info.num_subcores
    nw = nc * ns
    assert num_rows % (nw * CHUNK) == 0
    chunks_per_w = num_rows // (nw * CHUNK)
    assert chunks_per_w % (2 * WBUF) == 0
    ngroups = chunks_per_w // (2 * WBUF)
    mesh = plsc.VectorSubcoreMesh(core_axis_name="c", subcore_axis_name="s")

    @functools.partial(
        pl.kernel,
        mesh=mesh,
        out_type=jax.ShapeDtypeStruct((num_rows, EMBEDDING_DIM), jnp.float32),
        scratch_types=[
            pltpu.VMEM((chunks_per_w, CHUNK), jnp.int32),
            pltpu.VMEM((WBUF, 2 * CHUNK, EMBEDDING_DIM), jnp.float32),
            pltpu.SemaphoreType.DMA((WBUF, 2)),
            pltpu.SemaphoreType.DMA((WBUF,)),
        ],
    )
    def gather_k(table_hbm, idx_hbm, out_hbm, idx_v, rows_v, gsem, osem):
        wid = lax.axis_index("s") * nc + lax.axis_index("c")
        row0 = wid * chunks_per_w * CHUNK
        pltpu.sync_copy(idx_hbm.at[wid], idx_v)

        def gather(j, q, h):
            return pltpu.make_async_copy(
                table_hbm.at[idx_v.at[j]],
                rows_v.at[q, pl.ds(h * CHUNK, CHUNK)],
                gsem.at[q, h],
            )

        def copy_out(j, q):
            return pltpu.make_async_copy(
                rows_v.at[q],
                out_hbm.at[pl.ds(row0 + j * CHUNK, 2 * CHUNK)],
                osem.at[q],
            )

        for q in range(WBUF):
            gather(2 * q, q, 0).start()
            gather(2 * q + 1, q, 1).start()

        def body(g, _):
            j0 = g * 2 * WBUF
            for q in range(WBUF):
                gather(j0 + 2 * q, q, 0).wait()
                gather(j0 + 2 * q + 1, q, 1).wait()
                copy_out(j0 + 2 * q, q).start()

            @pl.when(g + 1 < ngroups)
            def _():
                for q in range(WBUF):
                    copy_out(j0 + 2 * q, q).wait()
                    gather(j0 + 2 * WBUF + 2 * q, q, 0).start()
                    gather(j0 + 2 * WBUF + 2 * q + 1, q, 1).start()

            return 0

        lax.fori_loop(0, ngroups, body, 0)
        for q in range(WBUF):
            copy_out((ngroups - 1) * 2 * WBUF + 2 * q, q).wait()

    return gather_k


def kernel(input_, weight):
    b, s = input_.shape
    idx = input_.reshape(-1).astype(jnp.int32)
    num_rows = idx.shape[0]
    info = plsc.get_sparse_core_info()
    nw = info.num_cores * info.num_subcores
    idx2d = idx.reshape(nw, num_rows // (nw * CHUNK), CHUNK)
    out = _make_gather(num_rows)(weight, idx2d)
    return out.reshape(b, s, EMBEDDING_DIM)


# speedup vs baseline: 1.0075x; 1.0075x over previous
import functools

import jax
import jax.numpy as jnp
from jax import lax
from jax.experimental import pallas as pl
from jax.experimental.pallas import tpu as pltpu
from jax.experimental.pallas import tpu_sc as plsc

EMBEDDING_DIM = 128
CHUNK = 64
NBUF = 4
RBUF = 6


def _make_gather(num_rows: int):
    info = plsc.get_sparse_core_info()
    nc, ns = info.num_cores, info.num_subcores
    nw = nc * ns
    assert num_rows % (nw * CHUNK) == 0
    chunks_per_w = num_rows // (nw * CHUNK)
    assert chunks_per_w % NBUF == 0
    ngroups = chunks_per_w // NBUF
    mesh = plsc.VectorSubcoreMesh(core_axis_name="c", subcore_axis_name="s")

    @functools.partial(
        pl.kernel,
        mesh=mesh,
        out_type=jax.ShapeDtypeStruct((num_rows, EMBEDDING_DIM), jnp.float32),
        scratch_types=[
            pltpu.VMEM((chunks_per_w, CHUNK), jnp.int32),
            pltpu.VMEM((NBUF, CHUNK, EMBEDDING_DIM), jnp.float32),
            pltpu.VMEM_SHARED((ns, RBUF, CHUNK, EMBEDDING_DIM), jnp.float32),
            pltpu.SemaphoreType.DMA((NBUF,)),
            pltpu.SemaphoreType.DMA((NBUF,)),
            pltpu.SemaphoreType.DMA((RBUF,)),
        ],
    )
    def gather_k(table_hbm, idx_hbm, out_hbm, idx_v, rows_v, stage, gsem, osem, rsem):
        sid = lax.axis_index("s")
        wid = sid * nc + lax.axis_index("c")
        row0 = wid * chunks_per_w * CHUNK
        pltpu.sync_copy(idx_hbm.at[wid], idx_v)

        def gather(j, b):
            return pltpu.make_async_copy(
                table_hbm.at[idx_v.at[j]], rows_v.at[b], gsem.at[b]
            )

        def copy_direct(j, b):
            return pltpu.make_async_copy(
                rows_v.at[b],
                out_hbm.at[pl.ds(row0 + j * CHUNK, CHUNK)],
                osem.at[b],
            )

        def copy_spmem_out(j, r):
            return pltpu.make_async_copy(
                stage.at[sid, r],
                out_hbm.at[pl.ds(row0 + j * CHUNK, CHUNK)],
                rsem.at[r],
            )

        for b in range(NBUF):
            gather(b, b).start()

        def body(g, _):
            j0 = g * NBUF
            for t in range(NBUF):
                b, j = t, j0 + t
                gather(j, b).wait()
                if t % 2 == 0:
                    copy_direct(j, b).start()

                    @pl.when(g + 1 < ngroups)
                    def _():
                        copy_direct(j, b).wait()
                        gather(j + NBUF, b).start()
                else:
                    o = 2 * g + (t // 2)
                    r = lax.rem(o, RBUF)

                    @pl.when(o >= RBUF)
                    def _():
                        copy_spmem_out(j, r).wait()

                    pltpu.sync_copy(rows_v.at[b], stage.at[sid, r])
                    copy_spmem_out(j, r).start()

                    @pl.when(g + 1 < ngroups)
                    def _():
                        gather(j + NBUF, b).start()

            return 0

        lax.fori_loop(0, ngroups, body, 0)
        for b in range(0, NBUF, 2):
            copy_direct((ngroups - 1) * NBUF + b, b).wait()
        n_odd = chunks_per_w // 2
        for r in range(RBUF):
            last_o = n_odd - 1 - ((n_odd - 1 - r) % RBUF)
            g_last = last_o // 2
            t_last = 1 + 2 * (last_o % 2)
            copy_spmem_out(g_last * NBUF + t_last, r).wait()

    return gather_k


def kernel(input_, weight):
    b, s = input_.shape
    idx = input_.reshape(-1).astype(jnp.int32)
    num_rows = idx.shape[0]
    info = plsc.get_sparse_core_info()
    nw = info.num_cores * info.num_subcores
    idx2d = idx.reshape(nw, num_rows // (nw * CHUNK), CHUNK)
    out = _make_gather(num_rows)(weight, idx2d)
    return out.reshape(b, s, EMBEDDING_DIM)


# speedup vs baseline: 1.0089x; 1.0014x over previous
import functools

import jax
import jax.numpy as jnp
from jax import lax
from jax.experimental import pallas as pl
from jax.experimental.pallas import tpu as pltpu
from jax.experimental.pallas import tpu_sc as plsc

EMBEDDING_DIM = 128
CHUNK = 64
NBUF = 4
RBUF = 9


def _make_gather(num_rows: int):
    info = plsc.get_sparse_core_info()
    nc, ns = info.num_cores, info.num_subcores
    nw = nc * ns
    assert num_rows % (nw * CHUNK) == 0
    chunks_per_w = num_rows // (nw * CHUNK)
    assert chunks_per_w % NBUF == 0
    ngroups = chunks_per_w // NBUF
    mesh = plsc.VectorSubcoreMesh(core_axis_name="c", subcore_axis_name="s")

    @functools.partial(
        pl.kernel,
        mesh=mesh,
        out_type=jax.ShapeDtypeStruct((num_rows, EMBEDDING_DIM), jnp.float32),
        scratch_types=[
            pltpu.VMEM((chunks_per_w, CHUNK), jnp.int32),
            pltpu.VMEM((NBUF, CHUNK, EMBEDDING_DIM), jnp.float32),
            pltpu.VMEM_SHARED((ns, RBUF, CHUNK, EMBEDDING_DIM), jnp.float32),
            pltpu.SemaphoreType.DMA((NBUF,)),
            pltpu.SemaphoreType.DMA((NBUF,)),
            pltpu.SemaphoreType.DMA((RBUF,)),
        ],
    )
    def gather_k(table_hbm, idx_hbm, out_hbm, idx_v, rows_v, stage, gsem, osem, rsem):
        sid = lax.axis_index("s")
        wid = sid * nc + lax.axis_index("c")
        row0 = wid * chunks_per_w * CHUNK
        pltpu.sync_copy(idx_hbm.at[wid], idx_v)

        def gather(j, b):
            return pltpu.make_async_copy(
                table_hbm.at[idx_v.at[j]], rows_v.at[b], gsem.at[b]
            )

        def copy_direct(j, b):
            return pltpu.make_async_copy(
                rows_v.at[b],
                out_hbm.at[pl.ds(row0 + j * CHUNK, CHUNK)],
                osem.at[b],
            )

        def copy_spmem_out(j, r):
            return pltpu.make_async_copy(
                stage.at[sid, r],
                out_hbm.at[pl.ds(row0 + j * CHUNK, CHUNK)],
                rsem.at[r],
            )

        for b in range(NBUF):
            gather(b, b).start()

        def body(g, _):
            j0 = g * NBUF
            for t in range(NBUF):
                b, j = t, j0 + t
                gather(j, b).wait()
                if t == 0:
                    copy_direct(j, b).start()

                    @pl.when(g + 1 < ngroups)
                    def _():
                        copy_direct(j, b).wait()
                        gather(j + NBUF, b).start()
                else:
                    o = 3 * g + (t - 1)
                    r = lax.rem(o, RBUF)

                    @pl.when(o >= RBUF)
                    def _():
                        copy_spmem_out(j, r).wait()

                    pltpu.sync_copy(rows_v.at[b], stage.at[sid, r])
                    copy_spmem_out(j, r).start()

                    @pl.when(g + 1 < ngroups)
                    def _():
                        gather(j + NBUF, b).start()

            return 0

        lax.fori_loop(0, ngroups, body, 0)
        copy_direct((ngroups - 1) * NBUF, 0).wait()
        n_staged = 3 * ngroups
        for r in range(RBUF):
            last_o = n_staged - 1 - ((n_staged - 1 - r) % RBUF)
            g_last = last_o // 3
            t_last = 1 + (last_o % 3)
            copy_spmem_out(g_last * NBUF + t_last, r).wait()

    return gather_k


def kernel(input_, weight):
    b, s = input_.shape
    idx = input_.reshape(-1).astype(jnp.int32)
    num_rows = idx.shape[0]
    info = plsc.get_sparse_core_info()
    nw = info.num_cores * info.num_subcores
    idx2d = idx.reshape(nw, num_rows // (nw * CHUNK), CHUNK)
    out = _make_gather(num_rows)(weight, idx2d)
    return out.reshape(b, s, EMBEDDING_DIM)


# speedup vs baseline: 1.0092x; 1.0003x over previous
import functools

import jax
import jax.numpy as jnp
from jax import lax
from jax.experimental import pallas as pl
from jax.experimental.pallas import tpu as pltpu
from jax.experimental.pallas import tpu_sc as plsc

EMBEDDING_DIM = 128
CHUNK = 64
NBUF = 5
RBUF = 8


def _make_gather(num_rows: int):
    info = plsc.get_sparse_core_info()
    nc, ns = info.num_cores, info.num_subcores
    nw = nc * ns
    assert num_rows % (nw * CHUNK) == 0
    chunks_per_w = num_rows // (nw * CHUNK)
    assert chunks_per_w % NBUF == 0
    ngroups = chunks_per_w // NBUF
    mesh = plsc.VectorSubcoreMesh(core_axis_name="c", subcore_axis_name="s")

    @functools.partial(
        pl.kernel,
        mesh=mesh,
        out_type=jax.ShapeDtypeStruct((num_rows, EMBEDDING_DIM), jnp.float32),
        scratch_types=[
            pltpu.VMEM((chunks_per_w, CHUNK), jnp.int32),
            pltpu.VMEM((NBUF, CHUNK, EMBEDDING_DIM), jnp.float32),
            pltpu.VMEM_SHARED((ns, RBUF, CHUNK, EMBEDDING_DIM), jnp.float32),
            pltpu.SemaphoreType.DMA((NBUF,)),
            pltpu.SemaphoreType.DMA((NBUF,)),
            pltpu.SemaphoreType.DMA((RBUF,)),
        ],
    )
    def gather_k(table_hbm, idx_hbm, out_hbm, idx_v, rows_v, stage, gsem, osem, rsem):
        sid = lax.axis_index("s")
        wid = sid * nc + lax.axis_index("c")
        row0 = wid * chunks_per_w * CHUNK
        pltpu.sync_copy(idx_hbm.at[wid], idx_v)

        def gather(j, b):
            return pltpu.make_async_copy(
                table_hbm.at[idx_v.at[j]], rows_v.at[b], gsem.at[b]
            )

        def copy_direct(j, b):
            return pltpu.make_async_copy(
                rows_v.at[b],
                out_hbm.at[pl.ds(row0 + j * CHUNK, CHUNK)],
                osem.at[b],
            )

        def copy_spmem_out(j, r):
            return pltpu.make_async_copy(
                stage.at[sid, r],
                out_hbm.at[pl.ds(row0 + j * CHUNK, CHUNK)],
                rsem.at[r],
            )

        for b in range(NBUF):
            gather(b, b).start()

        def body(g, _):
            j0 = g * NBUF
            for t in range(NBUF):
                b, j = t, j0 + t
                gather(j, b).wait()
                if t == 0:
                    copy_direct(j, b).start()

                    @pl.when(g + 1 < ngroups)
                    def _():
                        copy_direct(j, b).wait()
                        gather(j + NBUF, b).start()
                else:
                    o = 4 * g + (t - 1)
                    r = lax.rem(o, RBUF)

                    @pl.when(o >= RBUF)
                    def _():
                        copy_spmem_out(j, r).wait()

                    pltpu.sync_copy(rows_v.at[b], stage.at[sid, r])
                    copy_spmem_out(j, r).start()

                    @pl.when(g + 1 < ngroups)
                    def _():
                        gather(j + NBUF, b).start()

            return 0

        lax.fori_loop(0, ngroups, body, 0)
        copy_direct((ngroups - 1) * NBUF, 0).wait()
        n_staged = 4 * ngroups
        for r in range(RBUF):
            last_o = n_staged - 1 - ((n_staged - 1 - r) % RBUF)
            g_last = last_o // 4
            t_last = 1 + (last_o % 4)
            copy_spmem_out(g_last * NBUF + t_last, r).wait()

    return gather_k


def kernel(input_, weight):
    b, s = input_.shape
    idx = input_.reshape(-1).astype(jnp.int32)
    num_rows = idx.shape[0]
    info = plsc.get_sparse_core_info()
    nw = info.num_cores * info.num_subcores
    idx2d = idx.reshape(nw, num_rows // (nw * CHUNK), CHUNK)
    out = _make_gather(num_rows)(weight, idx2d)
    return out.reshape(b, s, EMBEDDING_DIM)
